# Initial kernel scaffold; baseline (speedup 1.0000x reference)
import jax
import jax.numpy as jnp
from jax.experimental import pallas as pl


def kernel(input_feature, edge_index, adj_values, W, b):
    raise NotImplementedError("write your pallas kernel here")


# speedup vs baseline: 3.2651x; 3.2651x over previous
import functools

import jax
import jax.numpy as jnp
from jax import lax
from jax.experimental import pallas as pl
from jax.experimental.pallas import tpu as pltpu
from jax.experimental.pallas import tpu_sc as plsc

N = 10000
NE = 320000
D = 128
TILES = 32
CK = 128
CHUNKS = 80
EPT = CHUNKS * CK
NE_PAD = TILES * EPT
N_ACC = 10240
RPT = N_ACC // 16


def _mm_body(x_ref, w_ref, o_ref):
    o_ref[...] = jnp.dot(x_ref[...], w_ref[...],
                         preferred_element_type=jnp.float32)


def _matmul(x, w):
    return pl.pallas_call(
        _mm_body,
        grid=(10,),
        in_specs=[
            pl.BlockSpec((N // 10, D), lambda i: (i, 0)),
            pl.BlockSpec((D, D), lambda i: (0, 0)),
        ],
        out_specs=pl.BlockSpec((N // 10, D), lambda i: (i, 0)),
        out_shape=jax.ShapeDtypeStruct((N, D), jnp.float32),
    )(x, w)


def _comb_body(p_ref, b_ref, o_ref):
    o_ref[...] = p_ref[0] + p_ref[1] + b_ref[...]


def _combine(partials, b2d):
    return pl.pallas_call(
        _comb_body,
        grid=(10,),
        in_specs=[
            pl.BlockSpec((2, N // 10, D), lambda i: (0, i, 0)),
            pl.BlockSpec((1, D), lambda i: (0, 0)),
        ],
        out_specs=pl.BlockSpec((N // 10, D), lambda i: (i, 0)),
        out_shape=jax.ShapeDtypeStruct((N, D), jnp.float32),
    )(partials, b2d)


_MESH = plsc.VectorSubcoreMesh(core_axis_name="c", subcore_axis_name="s")


@functools.partial(
    pl.kernel,
    mesh=_MESH,
    out_type=jax.ShapeDtypeStruct((2, N, D), jnp.float32),
    scratch_types=[
        pltpu.VMEM((CHUNKS, CK), jnp.int32),
        pltpu.VMEM((CHUNKS, CK), jnp.int32),
        pltpu.VMEM((CHUNKS, CK), jnp.float32),
        pltpu.VMEM((CK, D), jnp.float32),
        pltpu.VMEM_SHARED((N_ACC, D), jnp.float32),
        pltpu.SemaphoreType.DMA,
    ],
)
def _sc_aggregate(support_hbm, src_hbm, dst_hbm, adj_hbm, out_hbm,
                  src_v, dst_v, adj_v, rows, acc, sem):
    c = lax.axis_index("c")
    s = lax.axis_index("s")
    wid = s * 2 + c

    pltpu.sync_copy(src_hbm.at[wid], src_v)
    pltpu.sync_copy(dst_hbm.at[wid], dst_v)
    pltpu.sync_copy(adj_hbm.at[wid], adj_v)

    zf = jnp.zeros((16,), jnp.float32)

    def _zrow(i, carry):
        for g in range(8):
            rows[i, pl.ds(g * 16, 16)] = zf
        return carry

    lax.fori_loop(0, CK, _zrow, 0)
    r0 = s * RPT
    for k in range(RPT // CK):
        pltpu.sync_copy(rows, acc.at[pl.ds(r0 + k * CK, CK)])
    plsc.subcore_barrier()

    def _chunk(g, carry):
        pltpu.async_copy(support_hbm.at[src_v.at[g]], rows, sem).wait()

        def _grp(j16, ecarry):
            a16 = adj_v[g, pl.ds(j16 * 16, 16)]
            for l in range(16):
                j = j16 * 16 + l
                a = jnp.full((16,), a16[l], jnp.float32)
                for cg in range(8):
                    sl = pl.ds(cg * 16, 16)
                    rows[j, sl] = rows[j, sl] * a
            return ecarry

        lax.fori_loop(0, CK // 16, _grp, 0)
        pltpu.sync_copy(rows, acc.at[dst_v.at[g]], add=True)
        return carry

    lax.fori_loop(0, CHUNKS, _chunk, 0)
    plsc.subcore_barrier()

    @pl.when(s < 15)
    def _():
        pltpu.sync_copy(acc.at[pl.ds(r0, RPT)], out_hbm.at[c, pl.ds(r0, RPT)])

    @pl.when(s == 15)
    def _():
        last = N - 15 * RPT
        pltpu.sync_copy(acc.at[pl.ds(15 * RPT, last)],
                        out_hbm.at[c, pl.ds(15 * RPT, last)])


def kernel(input_feature, edge_index, adj_values, W, b):
    src = edge_index[0].astype(jnp.int32)
    dst = edge_index[1].astype(jnp.int32)
    adj = adj_values.astype(jnp.float32)
    pad = NE_PAD - NE
    src = jnp.concatenate([src, jnp.zeros((pad,), jnp.int32)])
    dst = jnp.concatenate([dst, jnp.zeros((pad,), jnp.int32)])
    adj = jnp.concatenate([adj, jnp.zeros((pad,), jnp.float32)])
    src = src.reshape(TILES, CHUNKS, CK)
    dst = dst.reshape(TILES, CHUNKS, CK)
    adj = adj.reshape(TILES, CHUNKS, CK)

    support = _matmul(input_feature.astype(jnp.float32), W.astype(jnp.float32))
    partials = _sc_aggregate(support, src, dst, adj)
    return _combine(partials, b.reshape(1, D).astype(jnp.float32))
